# baseline (speedup 1.0000x reference)
import functools

import jax
import jax.numpy as jnp
from jax import lax
from jax.experimental import pallas as pl
from jax.experimental.pallas import tpu as pltpu
from jax.experimental.pallas import tpu_sc as plsc

HASH_SIZE = 1000000
PROJ_DIM = 128
MODEL_DIM = 512
LANES = 16
N_CHUNKS = 1
MM_BLK = 1024


@functools.cache
def _sc_gather(n_chunk: int, batch: int, seq: int, offset: int):
    info = plsc.get_sparse_core_info()
    nc, ns = info.num_cores, info.num_subcores
    nw = nc * ns
    assert n_chunk % (nw * 128) == 0
    b_per_w = n_chunk // nw
    assert seq % b_per_w == 0
    n_gath = b_per_w // 128
    vec_per_g = 128 // LANES

    mesh = plsc.VectorSubcoreMesh(core_axis_name="c", subcore_axis_name="s")

    @functools.partial(
        pl.kernel,
        mesh=mesh,
        out_type=jax.ShapeDtypeStruct((n_chunk, PROJ_DIM), jnp.float32),
        scratch_types=[
            pltpu.VMEM((b_per_w + 24,), jnp.int32),
            pltpu.VMEM((n_gath, 128), jnp.int32),
            pltpu.VMEM((b_per_w, PROJ_DIM), jnp.float32),
            pltpu.SemaphoreType.DMA,
            pltpu.SemaphoreType.DMA,
        ],
    )
    def gather(t_hbm, table_hbm, out_hbm, tok_v, idx_v, rows_v, gsem, osem):
        wid = lax.axis_index("s") * nc + lax.axis_index("c")
        base = wid * b_per_w
        gbase = offset + base
        cstart = pl.multiple_of(jnp.maximum(gbase - 8, 0), 8)
        sh = 16 + jnp.minimum(gbase, 8)
        pltpu.sync_copy(t_hbm.at[pl.ds(cstart, b_per_w + 8)],
                        tok_v.at[pl.ds(16, b_per_w + 8)])
        lane = lax.iota(jnp.int32, LANES)
        gathers = []
        for g in range(n_gath):
            def hash_body(v, _, g=g):
                o = g * 128 + v * LANES
                cur = tok_v[pl.ds(sh + o, LANES)]
                prv = tok_v[pl.ds(sh + (o - 1), LANES)]
                h = jnp.mod(jnp.bitwise_xor(cur * 36313, prv * 27191),
                            HASH_SIZE - 1)
                pos = (gbase + o) + lane
                h = jnp.where(pos % seq == 0, HASH_SIZE - 1, h)
                idx_v[g, pl.ds(v * LANES, LANES)] = h
                return 0
            lax.fori_loop(0, vec_per_g, hash_body, 0)
            gathers.append(pltpu.async_copy(
                table_hbm.at[idx_v.at[g]],
                rows_v.at[pl.ds(g * 128, 128)],
                gsem,
            ))
        drains = []
        for g in range(n_gath):
            gathers[g].wait()
            drains.append(pltpu.async_copy(
                rows_v.at[pl.ds(g * 128, 128)],
                out_hbm.at[pl.ds(base + g * 128, 128)],
                osem,
            ))
        for d in drains:
            d.wait()

    return gather


def _mm_body(emb_ref, w_ref, scale_ref, out_ref):
    acc = lax.dot_general(
        emb_ref[...], w_ref[...],
        (((1,), (1,)), ((), ())),
        preferred_element_type=jnp.float32,
    )
    out_ref[...] = acc * scale_ref[0]


def _mm_body_acc(emb_ref, w_ref, scale_ref, prev_ref, out_ref):
    _mm_body(emb_ref, w_ref, scale_ref, out_ref)


@functools.cache
def _project(n_tokens: int, n_chunk: int, chunk: int, first: bool):
    blk = MM_BLK
    blk_off = chunk * (n_chunk // blk)
    in_specs = [
        pl.BlockSpec((blk, PROJ_DIM), lambda i: (i, 0)),
        pl.BlockSpec((MODEL_DIM, PROJ_DIM), lambda i: (0, 0)),
        pl.BlockSpec(memory_space=pltpu.SMEM),
    ]
    kwargs = {}
    body = _mm_body
    if not first:
        in_specs.append(pl.BlockSpec(memory_space=pl.ANY))
        kwargs["input_output_aliases"] = {3: 0}
        body = _mm_body_acc
    return pl.pallas_call(
        body,
        grid=(n_chunk // blk,),
        in_specs=in_specs,
        out_specs=pl.BlockSpec((blk, MODEL_DIM), lambda i: (i + blk_off, 0)),
        out_shape=jax.ShapeDtypeStruct((n_tokens, MODEL_DIM), jnp.float32),
        **kwargs,
    )


def kernel(token_ids, embed_weight, proj_weight, scale):
    b, s = token_ids.shape
    n = b * s
    n_chunk = n // N_CHUNKS
    t = token_ids.astype(jnp.int32).reshape(n)
    scale_arr = jnp.asarray(scale, jnp.float32).reshape(1)
    embs = [
        _sc_gather(n_chunk, b, s, c * n_chunk)(t, embed_weight)
        for c in range(N_CHUNKS)
    ]
    out = _project(n, n_chunk, 0, True)(embs[0], proj_weight, scale_arr)
    for c in range(1, N_CHUNKS):
        out = _project(n, n_chunk, c, False)(
            embs[c], proj_weight, scale_arr, out)
    return out.reshape(b, s, MODEL_DIM)


# speedup vs baseline: 1.1220x; 1.1220x over previous
import functools

import jax
import jax.numpy as jnp
from jax import lax
from jax.experimental import pallas as pl
from jax.experimental.pallas import tpu as pltpu
from jax.experimental.pallas import tpu_sc as plsc

HASH_SIZE = 1000000
PROJ_DIM = 128
MODEL_DIM = 512
LANES = 16
N_CHUNKS = 1
MM_BLK = 4096


@functools.cache
def _sc_gather(n_chunk: int, batch: int, seq: int, offset: int):
    info = plsc.get_sparse_core_info()
    nc, ns = info.num_cores, info.num_subcores
    nw = nc * ns
    assert n_chunk % (nw * 128) == 0
    b_per_w = n_chunk // nw
    assert seq % b_per_w == 0
    n_gath = b_per_w // 128
    vec_per_g = 128 // LANES

    mesh = plsc.VectorSubcoreMesh(core_axis_name="c", subcore_axis_name="s")

    @functools.partial(
        pl.kernel,
        mesh=mesh,
        out_type=jax.ShapeDtypeStruct((n_chunk, PROJ_DIM), jnp.float32),
        scratch_types=[
            pltpu.VMEM((b_per_w + 24,), jnp.int32),
            pltpu.VMEM((n_gath, 128), jnp.int32),
            pltpu.VMEM((b_per_w, PROJ_DIM), jnp.float32),
            pltpu.SemaphoreType.DMA,
            pltpu.SemaphoreType.DMA,
        ],
    )
    def gather(t_hbm, table_hbm, out_hbm, tok_v, idx_v, rows_v, gsem, osem):
        wid = lax.axis_index("s") * nc + lax.axis_index("c")
        base = wid * b_per_w
        gbase = offset + base
        cstart = pl.multiple_of(jnp.maximum(gbase - 8, 0), 8)
        sh = 16 + jnp.minimum(gbase, 8)
        pltpu.sync_copy(t_hbm.at[pl.ds(cstart, b_per_w + 8)],
                        tok_v.at[pl.ds(16, b_per_w + 8)])
        lane = lax.iota(jnp.int32, LANES)
        gathers = []
        for g in range(n_gath):
            def hash_body(v, _, g=g):
                o = g * 128 + v * LANES
                cur = tok_v[pl.ds(sh + o, LANES)]
                prv = tok_v[pl.ds(sh + (o - 1), LANES)]
                h = jnp.mod(jnp.bitwise_xor(cur * 36313, prv * 27191),
                            HASH_SIZE - 1)
                pos = (gbase + o) + lane
                h = jnp.where(pos % seq == 0, HASH_SIZE - 1, h)
                idx_v[g, pl.ds(v * LANES, LANES)] = h
                return 0
            lax.fori_loop(0, vec_per_g, hash_body, 0)
            gathers.append(pltpu.async_copy(
                table_hbm.at[idx_v.at[g]],
                rows_v.at[pl.ds(g * 128, 128)],
                gsem,
            ))
        drains = []
        for g in range(n_gath):
            gathers[g].wait()
            drains.append(pltpu.async_copy(
                rows_v.at[pl.ds(g * 128, 128)],
                out_hbm.at[pl.ds(base + g * 128, 128)],
                osem,
            ))
        for d in drains:
            d.wait()

    return gather


def _mm_body(emb_ref, w_ref, scale_ref, out_ref):
    acc = lax.dot_general(
        emb_ref[...], w_ref[...],
        (((1,), (1,)), ((), ())),
        preferred_element_type=jnp.float32,
    )
    out_ref[...] = acc * scale_ref[0]


def _mm_body_acc(emb_ref, w_ref, scale_ref, prev_ref, out_ref):
    _mm_body(emb_ref, w_ref, scale_ref, out_ref)


@functools.cache
def _project(n_tokens: int, n_chunk: int, chunk: int, first: bool):
    blk = MM_BLK
    blk_off = chunk * (n_chunk // blk)
    in_specs = [
        pl.BlockSpec((blk, PROJ_DIM), lambda i: (i, 0)),
        pl.BlockSpec((MODEL_DIM, PROJ_DIM), lambda i: (0, 0)),
        pl.BlockSpec(memory_space=pltpu.SMEM),
    ]
    kwargs = {}
    body = _mm_body
    if not first:
        in_specs.append(pl.BlockSpec(memory_space=pl.ANY))
        kwargs["input_output_aliases"] = {3: 0}
        body = _mm_body_acc
    return pl.pallas_call(
        body,
        grid=(n_chunk // blk,),
        in_specs=in_specs,
        out_specs=pl.BlockSpec((blk, MODEL_DIM), lambda i: (i + blk_off, 0)),
        out_shape=jax.ShapeDtypeStruct((n_tokens, MODEL_DIM), jnp.float32),
        **kwargs,
    )


def kernel(token_ids, embed_weight, proj_weight, scale):
    b, s = token_ids.shape
    n = b * s
    n_chunk = n // N_CHUNKS
    t = token_ids.astype(jnp.int32).reshape(n)
    scale_arr = jnp.asarray(scale, jnp.float32).reshape(1)
    embs = [
        _sc_gather(n_chunk, b, s, c * n_chunk)(t, embed_weight)
        for c in range(N_CHUNKS)
    ]
    out = _project(n, n_chunk, 0, True)(embs[0], proj_weight, scale_arr)
    for c in range(1, N_CHUNKS):
        out = _project(n, n_chunk, c, False)(
            embs[c], proj_weight, scale_arr, out)
    return out.reshape(b, s, MODEL_DIM)
